# baseline (speedup 1.0000x reference)
import jax
import jax.numpy as jnp
from jax import lax
from jax.experimental import pallas as pl
from jax.experimental.pallas import tpu as pltpu
from jax.experimental.pallas import tpu_sc as plsc

VOCAB = 6561
D = 1024
NB = 1024
T = 50
NC, NS = 2, 16
NW = NC * NS
RPW = NB // NW


def _emb_body(tokens_hbm, table_hbm, out_hbm, idx_v, buf0, buf1, gs0, gs1, os0, os1):
    wid = lax.axis_index("s") * NC + lax.axis_index("c")
    base = pl.multiple_of(wid * RPW, 8)
    pltpu.sync_copy(tokens_hbm.at[pl.ds(base, RPW)], idx_v)

    bufs = (buf0, buf1)
    gsems = (gs0, gs1)
    osems = (os0, os1)

    def start_gather(r, b):
        pltpu.async_copy(table_hbm.at[idx_v.at[r]], bufs[b], gsems[b])

    def wait_gather(b):
        pltpu.make_async_copy(table_hbm.at[pl.ds(0, T)], bufs[b], gsems[b]).wait()

    def start_scatter(r, b):
        pltpu.async_copy(bufs[b], out_hbm.at[base + r], osems[b])

    def wait_scatter(b):
        pltpu.make_async_copy(bufs[b], out_hbm.at[base], osems[b]).wait()

    start_gather(0, 0)

    @pl.loop(0, RPW // 2)
    def _pair(s):
        r0 = s * 2

        @pl.when(s > 0)
        def _():
            wait_scatter(1)
        start_gather(r0 + 1, 1)
        wait_gather(0)
        start_scatter(r0, 0)

        @pl.when(s < RPW // 2 - 1)
        def _():
            wait_scatter(0)
            start_gather(r0 + 2, 0)
        wait_gather(1)
        start_scatter(r0 + 1, 1)

    wait_scatter(0)
    wait_scatter(1)


@jax.jit
def _emb(tokens, table):
    run = pl.kernel(
        _emb_body,
        out_type=jax.ShapeDtypeStruct((NB, T, D), jnp.float32),
        mesh=plsc.VectorSubcoreMesh(core_axis_name="c", subcore_axis_name="s"),
        compiler_params=pltpu.CompilerParams(use_tc_tiling_on_sc=False),
        scratch_types=[
            pltpu.VMEM((RPW, T), jnp.int32),
            pltpu.VMEM((T, D), jnp.float32),
            pltpu.VMEM((T, D), jnp.float32),
            pltpu.SemaphoreType.DMA,
            pltpu.SemaphoreType.DMA,
            pltpu.SemaphoreType.DMA,
            pltpu.SemaphoreType.DMA,
        ],
    )
    return run(tokens, table)


def kernel(tokens, table):
    return _emb(tokens.astype(jnp.int32), table)


# speedup vs baseline: 1.1153x; 1.1153x over previous
import functools

import jax
import jax.numpy as jnp
from jax import lax
from jax.experimental import pallas as pl
from jax.experimental.pallas import tpu as pltpu
from jax.experimental.pallas import tpu_sc as plsc

VOCAB = 6561
D = 1024
B = 1024 * 50
NC, NS = 2, 16
NW = NC * NS
BPW = B // NW
C = 40
NCHUNK = BPW // C


def _emb_body(tokens_hbm, table_hbm, out_hbm, idx_v, buf, gsem, osem):
    wid = lax.axis_index("s") * NC + lax.axis_index("c")
    base = pl.multiple_of(wid * BPW, 8)
    pltpu.sync_copy(tokens_hbm.at[pl.ds(base, BPW)], idx_v)

    @pl.loop(0, NCHUNK)
    def _chunk(g):
        off = pl.multiple_of(g * C, 8)
        pltpu.async_copy(table_hbm.at[idx_v.at[pl.ds(off, C)]], buf, gsem).wait()
        pltpu.async_copy(buf, out_hbm.at[pl.ds(base + off, C)], osem).wait()


@jax.jit
def _emb(tokens_flat, table):
    run = pl.kernel(
        _emb_body,
        out_type=jax.ShapeDtypeStruct((B, D), jnp.float32),
        mesh=plsc.VectorSubcoreMesh(core_axis_name="c", subcore_axis_name="s"),
        scratch_types=[
            pltpu.VMEM((BPW,), jnp.int32),
            pltpu.VMEM((C, D), jnp.float32),
            pltpu.SemaphoreType.DMA,
            pltpu.SemaphoreType.DMA,
        ],
    )
    return run(tokens_flat, table)


def kernel(tokens, table):
    bt = tokens.shape
    out = _emb(tokens.reshape(-1).astype(jnp.int32), table)
    return out.reshape(*bt, D)


# speedup vs baseline: 1.1580x; 1.0382x over previous
import jax
import jax.numpy as jnp
from jax import lax
from jax.experimental import pallas as pl
from jax.experimental.pallas import tpu as pltpu
from jax.experimental.pallas import tpu_sc as plsc

VOCAB = 6561
D = 1024
B = 1024 * 50
NC, NS = 2, 16
NW = NC * NS
BPW = B // NW
C = 40
NCHUNK = BPW // C
NPAIR = NCHUNK // 2


def _emb_body(tokens_hbm, table_hbm, out_hbm, idx_v, buf0, buf1, gs0, gs1, os0, os1):
    wid = lax.axis_index("s") * NC + lax.axis_index("c")
    base = pl.multiple_of(wid * BPW, 8)
    pltpu.sync_copy(tokens_hbm.at[pl.ds(base, BPW)], idx_v)

    def gather(c, buf, sem):
        off = pl.multiple_of(c * C, 8)
        pltpu.async_copy(table_hbm.at[idx_v.at[pl.ds(off, C)]], buf, sem)

    def scatter(c, buf, sem):
        off = pl.multiple_of(c * C, 8)
        pltpu.async_copy(buf, out_hbm.at[pl.ds(base + off, C)], sem)

    def gwait(buf, sem):
        pltpu.make_async_copy(table_hbm.at[pl.ds(0, C)], buf, sem).wait()

    def swait(buf, sem):
        pltpu.make_async_copy(buf, out_hbm.at[pl.ds(base, C)], sem).wait()

    gather(0, buf0, gs0)
    gather(1, buf1, gs1)

    @pl.loop(0, NPAIR - 1)
    def _step(s):
        c0 = s * 2
        gwait(buf0, gs0)
        scatter(c0, buf0, os0)
        gwait(buf1, gs1)
        scatter(c0 + 1, buf1, os1)
        swait(buf0, os0)
        gather(c0 + 2, buf0, gs0)
        swait(buf1, os1)
        gather(c0 + 3, buf1, gs1)

    gwait(buf0, gs0)
    scatter(NCHUNK - 2, buf0, os0)
    gwait(buf1, gs1)
    scatter(NCHUNK - 1, buf1, os1)
    swait(buf0, os0)
    swait(buf1, os1)


@jax.jit
def _emb(tokens_flat, table):
    run = pl.kernel(
        _emb_body,
        out_type=jax.ShapeDtypeStruct((B, D), jnp.float32),
        mesh=plsc.VectorSubcoreMesh(core_axis_name="c", subcore_axis_name="s"),
        scratch_types=[
            pltpu.VMEM((BPW,), jnp.int32),
            pltpu.VMEM((C, D), jnp.float32),
            pltpu.VMEM((C, D), jnp.float32),
            pltpu.SemaphoreType.DMA,
            pltpu.SemaphoreType.DMA,
            pltpu.SemaphoreType.DMA,
            pltpu.SemaphoreType.DMA,
        ],
    )
    return run(tokens_flat, table)


def kernel(tokens, table):
    bt = tokens.shape
    out = _emb(tokens.reshape(-1).astype(jnp.int32), table)
    return out.reshape(*bt, D)


# speedup vs baseline: 3.2649x; 2.8196x over previous
import jax
import jax.numpy as jnp
from jax import lax
from jax.experimental import pallas as pl
from jax.experimental.pallas import tpu as pltpu
from jax.experimental.pallas import tpu_sc as plsc

VOCAB = 6561
D = 1024
B = 1024 * 50
NC, NS = 2, 16
NW = NC * NS
BPW = B // NW
C = 40
NCHUNK = BPW // C
NPAIR = NCHUNK // 2


def _emb_body(tokens_hbm, table_hbm, out_hbm, idx_v, buf0, buf1, gs0, gs1, os0, os1):
    wid = lax.axis_index("s") * NC + lax.axis_index("c")
    base = pl.multiple_of(wid * BPW, 8)
    pltpu.sync_copy(tokens_hbm.at[pl.ds(base, BPW)], idx_v)

    def gather(c, buf, sem):
        off = pl.multiple_of(c * C, 8)
        pltpu.async_copy(table_hbm.at[idx_v.at[pl.ds(off, C)]], buf, sem)

    def scatter(c, buf, sem):
        off = pl.multiple_of(c * C, 8)
        pltpu.async_copy(buf, out_hbm.at[pl.ds(base + off, C)], sem)

    def gwait(buf, sem):
        pltpu.make_async_copy(table_hbm.at[pl.ds(0, C)], buf, sem).wait()

    def swait(buf, sem):
        pltpu.make_async_copy(buf, out_hbm.at[pl.ds(base, C)], sem).wait()

    gather(0, buf0, gs0)
    gather(1, buf1, gs1)

    @pl.loop(0, NPAIR - 1)
    def _step(s):
        c0 = s * 2
        gwait(buf0, gs0)
        scatter(c0, buf0, os0)
        gwait(buf1, gs1)
        scatter(c0 + 1, buf1, os1)
        swait(buf0, os0)
        gather(c0 + 2, buf0, gs0)
        swait(buf1, os1)
        gather(c0 + 3, buf1, gs1)

    gwait(buf0, gs0)
    scatter(NCHUNK - 2, buf0, os0)
    gwait(buf1, gs1)
    scatter(NCHUNK - 1, buf1, os1)
    swait(buf0, os0)
    swait(buf1, os1)


@jax.jit
def _emb(tokens_flat, table):
    run = pl.kernel(
        _emb_body,
        out_type=jax.ShapeDtypeStruct((B, D), jnp.float32),
        mesh=plsc.VectorSubcoreMesh(core_axis_name="c", subcore_axis_name="s"),
        scratch_types=[
            pltpu.VMEM((BPW,), jnp.int32),
            pltpu.VMEM((C, D), jnp.float32),
            pltpu.VMEM((C, D), jnp.float32),
            pltpu.SemaphoreType.DMA,
            pltpu.SemaphoreType.DMA,
            pltpu.SemaphoreType.DMA,
            pltpu.SemaphoreType.DMA,
        ],
    )
    return run(tokens_flat, table)


def kernel(tokens, table):
    nb, t = tokens.shape
    tok_flat = tokens.T.reshape(-1).astype(jnp.int32)
    out = _emb(tok_flat, table)
    return out.reshape(t, nb, D).transpose(1, 0, 2)


# speedup vs baseline: 3.3115x; 1.0143x over previous
import jax
import jax.numpy as jnp
from jax import lax
from jax.experimental import pallas as pl
from jax.experimental.pallas import tpu as pltpu
from jax.experimental.pallas import tpu_sc as plsc

VOCAB = 6561
D = 1024
B = 1024 * 50
NC, NS = 2, 16
NW = NC * NS
BPW = B // NW
C = 16
NBUF = 4
NCHUNK = BPW // C
NGROUP = NCHUNK // NBUF


def _emb_body(tokens_hbm, table_hbm, out_hbm, idx_v,
              b0, b1, b2, b3, g0, g1, g2, g3, o0, o1, o2, o3):
    wid = lax.axis_index("s") * NC + lax.axis_index("c")
    base = pl.multiple_of(wid * BPW, 8)
    pltpu.sync_copy(tokens_hbm.at[pl.ds(base, BPW)], idx_v)

    bufs = (b0, b1, b2, b3)
    gsems = (g0, g1, g2, g3)
    osems = (o0, o1, o2, o3)

    def gather(c, buf, sem):
        off = pl.multiple_of(c * C, 8)
        pltpu.async_copy(table_hbm.at[idx_v.at[pl.ds(off, C)]], buf, sem)

    def scatter(c, buf, sem):
        off = pl.multiple_of(c * C, 8)
        pltpu.async_copy(buf, out_hbm.at[pl.ds(base + off, C)], sem)

    def gwait(buf, sem):
        pltpu.make_async_copy(out_hbm.at[pl.ds(base, C)], buf, sem).wait()

    def swait(buf, sem):
        pltpu.make_async_copy(buf, out_hbm.at[pl.ds(base, C)], sem).wait()

    for b in range(NBUF):
        gather(b, bufs[b], gsems[b])

    @pl.loop(0, NGROUP - 1)
    def _step(s):
        c0 = s * NBUF
        for b in range(NBUF):
            gwait(bufs[b], gsems[b])
            scatter(c0 + b, bufs[b], osems[b])
        for b in range(NBUF):
            swait(bufs[b], osems[b])
            gather(c0 + NBUF + b, bufs[b], gsems[b])

    cl = NCHUNK - NBUF
    for b in range(NBUF):
        gwait(bufs[b], gsems[b])
        scatter(cl + b, bufs[b], osems[b])
    for b in range(NBUF):
        swait(bufs[b], osems[b])


@jax.jit
def _emb(tokens_flat, table):
    run = pl.kernel(
        _emb_body,
        out_type=jax.ShapeDtypeStruct((B, D), jnp.float32),
        mesh=plsc.VectorSubcoreMesh(core_axis_name="c", subcore_axis_name="s"),
        scratch_types=[
            pltpu.VMEM((BPW,), jnp.int32),
        ] + [pltpu.VMEM((C, D), jnp.float32)] * 4
          + [pltpu.SemaphoreType.DMA] * 8,
    )
    return run(tokens_flat, table)


def kernel(tokens, table):
    nb, t = tokens.shape
    tok_flat = tokens.T.reshape(-1).astype(jnp.int32)
    out = _emb(tok_flat, table)
    return out.reshape(t, nb, D).transpose(1, 0, 2)


# speedup vs baseline: 3.3121x; 1.0002x over previous
import jax
import jax.numpy as jnp
from jax import lax
from jax.experimental import pallas as pl
from jax.experimental.pallas import tpu as pltpu
from jax.experimental.pallas import tpu_sc as plsc

VOCAB = 6561
D = 1024
B = 1024 * 50
NC, NS = 2, 16
NW = NC * NS
BPW = B // NW
C = 16
NBUF = 5
NCHUNK = BPW // C
NGROUP = NCHUNK // NBUF


def _emb_body(tokens_hbm, table_hbm, out_hbm, idx_v, *scratch):
    wid = lax.axis_index("s") * NC + lax.axis_index("c")
    base = pl.multiple_of(wid * BPW, 8)
    pltpu.sync_copy(tokens_hbm.at[pl.ds(base, BPW)], idx_v)

    bufs = scratch[:NBUF]
    gsems = scratch[NBUF:2 * NBUF]
    osems = scratch[2 * NBUF:]

    def gather(c, buf, sem):
        off = pl.multiple_of(c * C, 8)
        pltpu.async_copy(table_hbm.at[idx_v.at[pl.ds(off, C)]], buf, sem)

    def scatter(c, buf, sem):
        off = pl.multiple_of(c * C, 8)
        pltpu.async_copy(buf, out_hbm.at[pl.ds(base + off, C)], sem)

    def gwait(buf, sem):
        pltpu.make_async_copy(out_hbm.at[pl.ds(base, C)], buf, sem).wait()

    def swait(buf, sem):
        pltpu.make_async_copy(buf, out_hbm.at[pl.ds(base, C)], sem).wait()

    for b in range(NBUF):
        gather(b, bufs[b], gsems[b])

    @pl.loop(0, NGROUP - 1)
    def _step(s):
        c0 = s * NBUF
        for b in range(NBUF):
            gwait(bufs[b], gsems[b])
            scatter(c0 + b, bufs[b], osems[b])
        for b in range(NBUF):
            swait(bufs[b], osems[b])
            gather(c0 + NBUF + b, bufs[b], gsems[b])

    cl = NCHUNK - NBUF
    for b in range(NBUF):
        gwait(bufs[b], gsems[b])
        scatter(cl + b, bufs[b], osems[b])
    for b in range(NBUF):
        swait(bufs[b], osems[b])


@jax.jit
def _emb(tokens_flat, table):
    run = pl.kernel(
        _emb_body,
        out_type=jax.ShapeDtypeStruct((B, D), jnp.float32),
        mesh=plsc.VectorSubcoreMesh(core_axis_name="c", subcore_axis_name="s"),
        scratch_types=[
            pltpu.VMEM((BPW,), jnp.int32),
        ] + [pltpu.VMEM((C, D), jnp.float32)] * NBUF
          + [pltpu.SemaphoreType.DMA] * (2 * NBUF),
    )
    return run(tokens_flat, table)


def kernel(tokens, table):
    nb, t = tokens.shape
    tok_flat = tokens.T.reshape(-1).astype(jnp.int32)
    out = _emb(tok_flat, table)
    return out.reshape(t, nb, D).transpose(1, 0, 2)
